# baseline (speedup 1.0000x reference)
import functools

import jax
import jax.numpy as jnp
from jax import lax
from jax.experimental import pallas as pl
from jax.experimental.pallas import tpu as pltpu
from jax.experimental.pallas import tpu_sc as plsc

VOCAB = 1000000
EMBED = 64
B = 4096
L = 200

NC = 2
NS = 16
NW = NC * NS
BBLK = B // NW
NBUF = 4
GROUPS = EMBED // 16


def _body(idx_hbm, word_hbm, pos_hbm, out_hbm,
          idx_v, pos_v,
          buf0, buf1, buf2, buf3,
          obuf0, obuf1, obuf2, obuf3,
          gsem0, gsem1, gsem2, gsem3,
          osem0, osem1, osem2, osem3):
    bufs = (buf0, buf1, buf2, buf3)
    obufs = (obuf0, obuf1, obuf2, obuf3)
    gsems = (gsem0, gsem1, gsem2, gsem3)
    osems = (osem0, osem1, osem2, osem3)
    wid = lax.axis_index("s") * NC + lax.axis_index("c")

    pltpu.sync_copy(idx_hbm.at[:, pl.ds(wid * BBLK, BBLK)], idx_v)
    pltpu.sync_copy(pos_hbm.at[pl.ds(0, L)], pos_v)


    def start_gather(l, b):
        pltpu.async_copy(word_hbm.at[idx_v.at[l]], bufs[b], gsems[b])

    def wait_gather(l, b):
        pltpu.make_async_copy(word_hbm.at[idx_v.at[0]], bufs[b], gsems[b]).wait()

    def start_out(l, b):
        pltpu.async_copy(obufs[b], out_hbm.at[l, :, wid, :, :], osems[b])

    def wait_out(b):
        pltpu.make_async_copy(obufs[b], out_hbm.at[0, :, wid, :, :], osems[b]).wait()

    def transpose_add(l, b):
        buf, obuf = bufs[b], obufs[b]
        def bbody(r, _):
            lane = lax.iota(jnp.int32, 16)
            se0 = lane % 8
            lb = jnp.broadcast_to(r, (16,))
            for u in range(GROUPS):
                v = buf[r, pl.ds(u * 16, 16)] + pos_v[l, pl.ds(u * 16, 16)]
                plsc.store_scatter(obuf, [(lane + u * 16) // 8, se0, lb], v)
            return 0

        lax.fori_loop(0, BBLK, bbody, 0)

    for b in range(NBUF):
        start_gather(b, b)

    for b in range(NBUF):
        wait_gather(b, b)
        transpose_add(b, b)
        start_out(b, b)
        start_gather(b + NBUF, b)

    def outer(o, _):
        l0 = o * NBUF
        for b in range(NBUF):
            l = l0 + b
            wait_gather(l, b)
            wait_out(b)
            transpose_add(l, b)
            start_out(l, b)

            @pl.when(l + NBUF < L)
            def _():
                start_gather(l + NBUF, b)
        return 0

    lax.fori_loop(1, L // NBUF, outer, 0)
    for b in range(NBUF):
        wait_out(b)


_emb = functools.partial(
    pl.kernel,
    out_type=jax.ShapeDtypeStruct((L, EMBED // 8, B // 128, 8, 128),
                                  jnp.float32),
    mesh=plsc.VectorSubcoreMesh(core_axis_name="c", subcore_axis_name="s"),
    scratch_types=[
        pltpu.VMEM((L, BBLK), jnp.int32),
        pltpu.VMEM((L, EMBED), jnp.float32),
        pltpu.VMEM((BBLK, EMBED), jnp.float32),
        pltpu.VMEM((BBLK, EMBED), jnp.float32),
        pltpu.VMEM((BBLK, EMBED), jnp.float32),
        pltpu.VMEM((BBLK, EMBED), jnp.float32),
        pltpu.VMEM((EMBED // 8, 8, 128), jnp.float32),
        pltpu.VMEM((EMBED // 8, 8, 128), jnp.float32),
        pltpu.VMEM((EMBED // 8, 8, 128), jnp.float32),
        pltpu.VMEM((EMBED // 8, 8, 128), jnp.float32),
        pltpu.SemaphoreType.DMA,
        pltpu.SemaphoreType.DMA,
        pltpu.SemaphoreType.DMA,
        pltpu.SemaphoreType.DMA,
        pltpu.SemaphoreType.DMA,
        pltpu.SemaphoreType.DMA,
        pltpu.SemaphoreType.DMA,
        pltpu.SemaphoreType.DMA,
    ],
    compiler_params=pltpu.CompilerParams(use_tc_tiling_on_sc=False,
                                         needs_layout_passes=False),
)(_body)


def kernel(inputs, word_table, pos_table):
    idx_t = inputs.T
    out5 = _emb(idx_t, word_table, pos_table)
    out = out5.transpose(2, 4, 0, 1, 3).reshape(B, L, EMBED)
    return out


# speedup vs baseline: 1.0252x; 1.0252x over previous
import functools

import jax
import jax.numpy as jnp
from jax import lax
from jax.experimental import pallas as pl
from jax.experimental.pallas import tpu as pltpu
from jax.experimental.pallas import tpu_sc as plsc

VOCAB = 1000000
EMBED = 64
B = 4096
L = 200

NC = 2
NS = 16
NW = NC * NS
BBLK = B // NW
NBUF = 4
GROUPS = EMBED // 16
ROWL = 8 * 128
OUTW = (EMBED // 8) * (B // 128) * ROWL


def _body(idx_hbm, word_hbm, pos_hbm, out_hbm,
          idx_v, pos_v,
          buf0, buf1, buf2, buf3,
          obuf0, obuf1, obuf2, obuf3,
          gsem0, gsem1, gsem2, gsem3,
          osem0, osem1, osem2, osem3):
    bufs = (buf0, buf1, buf2, buf3)
    obufs = (obuf0, obuf1, obuf2, obuf3)
    gsems = (gsem0, gsem1, gsem2, gsem3)
    osems = (osem0, osem1, osem2, osem3)
    wid = lax.axis_index("s") * NC + lax.axis_index("c")

    pltpu.sync_copy(idx_hbm.at[:, pl.ds(wid * BBLK, BBLK)], idx_v)
    pltpu.sync_copy(pos_hbm.at[pl.ds(0, L)], pos_v)

    lane = lax.iota(jnp.int32, 16)
    base_u = [lane * 128 + u * 2048 for u in range(GROUPS)]

    def start_gather(l, b):
        pltpu.async_copy(word_hbm.at[idx_v.at[l]], bufs[b], gsems[b])

    def wait_gather(b):
        pltpu.make_async_copy(word_hbm.at[idx_v.at[0]], bufs[b], gsems[b]).wait()

    def start_out(l, b):
        for te in range(EMBED // 8):
            pltpu.async_copy(
                obufs[b].at[pl.ds(te * ROWL, ROWL)],
                out_hbm.at[l, pl.ds(te * 32 * ROWL + wid * ROWL, ROWL)],
                osems[b])

    def wait_out(b):
        for te in range(EMBED // 8):
            pltpu.make_async_copy(
                obufs[b].at[pl.ds(te * ROWL, ROWL)],
                out_hbm.at[0, pl.ds(te * 32 * ROWL, ROWL)],
                osems[b]).wait()

    def transpose_add(l, b):
        buf, obuf = bufs[b], obufs[b]
        pos_vecs = [pos_v[l, pl.ds(u * 16, 16)] for u in range(GROUPS)]

        def bbody(r, _):
            rb = jnp.broadcast_to(r, (16,))
            for u in range(GROUPS):
                v = buf[r, pl.ds(u * 16, 16)] + pos_vecs[u]
                plsc.store_scatter(obuf, [base_u[u] + rb], v)
            return 0

        lax.fori_loop(0, BBLK, bbody, 0, unroll=4)

    for b in range(NBUF):
        start_gather(b, b)

    for b in range(NBUF):
        wait_gather(b)
        transpose_add(b, b)
        start_out(b, b)
        start_gather(b + NBUF, b)

    def outer(o, _):
        l0 = o * NBUF
        for b in range(NBUF):
            l = l0 + b
            wait_gather(b)
            wait_out(b)
            transpose_add(l, b)
            start_out(l, b)

            @pl.when(l + NBUF < L)
            def _():
                start_gather(l + NBUF, b)
        return 0

    lax.fori_loop(1, L // NBUF, outer, 0)
    for b in range(NBUF):
        wait_out(b)


_emb = functools.partial(
    pl.kernel,
    out_type=jax.ShapeDtypeStruct((L, OUTW), jnp.float32),
    mesh=plsc.VectorSubcoreMesh(core_axis_name="c", subcore_axis_name="s"),
    scratch_types=[
        pltpu.VMEM((L, BBLK), jnp.int32),
        pltpu.VMEM((L, EMBED), jnp.float32),
        pltpu.VMEM((BBLK, EMBED), jnp.float32),
        pltpu.VMEM((BBLK, EMBED), jnp.float32),
        pltpu.VMEM((BBLK, EMBED), jnp.float32),
        pltpu.VMEM((BBLK, EMBED), jnp.float32),
        pltpu.VMEM((BBLK * EMBED,), jnp.float32),
        pltpu.VMEM((BBLK * EMBED,), jnp.float32),
        pltpu.VMEM((BBLK * EMBED,), jnp.float32),
        pltpu.VMEM((BBLK * EMBED,), jnp.float32),
        pltpu.SemaphoreType.DMA,
        pltpu.SemaphoreType.DMA,
        pltpu.SemaphoreType.DMA,
        pltpu.SemaphoreType.DMA,
        pltpu.SemaphoreType.DMA,
        pltpu.SemaphoreType.DMA,
        pltpu.SemaphoreType.DMA,
        pltpu.SemaphoreType.DMA,
    ],
    compiler_params=pltpu.CompilerParams(use_tc_tiling_on_sc=False,
                                         needs_layout_passes=False),
)(_body)


def kernel(inputs, word_table, pos_table):
    idx_t = inputs.T
    out2 = _emb(idx_t, word_table, pos_table)
    out5 = out2.reshape(L, EMBED // 8, B // 128, 8, 128)
    out = out5.transpose(2, 4, 0, 1, 3).reshape(B, L, EMBED)
    return out


# speedup vs baseline: 1.5865x; 1.5475x over previous
import functools

import jax
import jax.numpy as jnp
from jax import lax
from jax.experimental import pallas as pl
from jax.experimental.pallas import tpu as pltpu
from jax.experimental.pallas import tpu_sc as plsc

VOCAB = 1000000
EMBED = 64
B = 4096
L = 200

NC = 2
NS = 16
NW = NC * NS
BBLK = B // NW
NBUF = 4
GROUPS = EMBED // 16
ROWL = 8 * 128
OUTW = (EMBED // 8) * (B // 128) * ROWL


def _body(idx_hbm, word_hbm, pos_hbm, out_hbm,
          idx_v, pos_v,
          buf0, buf1, buf2, buf3,
          obuf0, obuf1, obuf2, obuf3,
          gsem0, gsem1, gsem2, gsem3,
          osem0, osem1, osem2, osem3):
    bufs = (buf0, buf1, buf2, buf3)
    obufs = (obuf0, obuf1, obuf2, obuf3)
    gsems = (gsem0, gsem1, gsem2, gsem3)
    osems = (osem0, osem1, osem2, osem3)
    wid = lax.axis_index("s") * NC + lax.axis_index("c")

    pltpu.sync_copy(idx_hbm.at[:, pl.ds(wid * BBLK, BBLK)], idx_v)
    pltpu.sync_copy(pos_hbm.at[pl.ds(0, L)], pos_v)

    lane = lax.iota(jnp.int32, 16)
    e_u = [lane + u * 16 for u in range(GROUPS)]

    def start_gather(l, b):
        pltpu.async_copy(word_hbm.at[idx_v.at[l]], bufs[b], gsems[b])

    def wait_gather(b):
        pltpu.make_async_copy(word_hbm.at[idx_v.at[0]], bufs[b], gsems[b]).wait()

    def start_out(l, b):
        for te in range(EMBED // 8):
            pltpu.async_copy(
                obufs[b].at[pl.ds(te * 8, 8), pl.ds(0, 128)],
                out_hbm.at[l, te * (B // 128) + wid],
                osems[b])

    def wait_out(b):
        for te in range(EMBED // 8):
            pltpu.make_async_copy(
                obufs[b].at[pl.ds(te * 8, 8), pl.ds(0, 128)],
                out_hbm.at[0, 0],
                osems[b]).wait()

    def transpose_add(l, b):
        buf, obuf = bufs[b], obufs[b]
        pos_vecs = [pos_v[l, pl.ds(u * 16, 16)] for u in range(GROUPS)]

        def bbody(r, _):
            rb = jnp.broadcast_to(r, (16,))
            for u in range(GROUPS):
                v = buf[r, pl.ds(u * 16, 16)] + pos_vecs[u]
                plsc.store_scatter(obuf, [e_u[u], rb], v)
            return 0

        lax.fori_loop(0, BBLK, bbody, 0, unroll=4)

    for b in range(NBUF):
        start_gather(b, b)

    for b in range(NBUF):
        wait_gather(b)
        transpose_add(b, b)
        start_out(b, b)
        start_gather(b + NBUF, b)

    def outer(o, _):
        l0 = o * NBUF
        for b in range(NBUF):
            l = l0 + b
            wait_gather(b)
            wait_out(b)
            transpose_add(l, b)
            start_out(l, b)

            @pl.when(l + NBUF < L)
            def _():
                start_gather(l + NBUF, b)
        return 0

    lax.fori_loop(1, L // NBUF, outer, 0)
    for b in range(NBUF):
        wait_out(b)


_emb = functools.partial(
    pl.kernel,
    out_type=jax.ShapeDtypeStruct((L, (EMBED // 8) * (B // 128), 8, 128),
                                  jnp.float32),
    mesh=plsc.VectorSubcoreMesh(core_axis_name="c", subcore_axis_name="s"),
    scratch_types=[
        pltpu.VMEM((L, BBLK), jnp.int32),
        pltpu.VMEM((L, EMBED), jnp.float32),
        pltpu.VMEM((BBLK, EMBED), jnp.float32),
        pltpu.VMEM((BBLK, EMBED), jnp.float32),
        pltpu.VMEM((BBLK, EMBED), jnp.float32),
        pltpu.VMEM((BBLK, EMBED), jnp.float32),
        pltpu.VMEM((EMBED, 129), jnp.float32),
        pltpu.VMEM((EMBED, 129), jnp.float32),
        pltpu.VMEM((EMBED, 129), jnp.float32),
        pltpu.VMEM((EMBED, 129), jnp.float32),
        pltpu.SemaphoreType.DMA,
        pltpu.SemaphoreType.DMA,
        pltpu.SemaphoreType.DMA,
        pltpu.SemaphoreType.DMA,
        pltpu.SemaphoreType.DMA,
        pltpu.SemaphoreType.DMA,
        pltpu.SemaphoreType.DMA,
        pltpu.SemaphoreType.DMA,
    ],
    compiler_params=pltpu.CompilerParams(use_tc_tiling_on_sc=False,
                                         needs_layout_passes=False),
)(_body)


def kernel(inputs, word_table, pos_table):
    idx_t = inputs.T
    out4 = _emb(idx_t, word_table, pos_table)
    out5 = out4.reshape(L, EMBED // 8, B // 128, 8, 128)
    out = out5.transpose(2, 4, 0, 1, 3).reshape(B, L, EMBED)
    return out
